# Initial kernel scaffold; baseline (speedup 1.0000x reference)
import jax
import jax.numpy as jnp
from jax.experimental import pallas as pl


def kernel(entity_vectors, query_entities, k):
    raise NotImplementedError("write your pallas kernel here")


# speedup vs baseline: 5.2551x; 5.2551x over previous
import functools

import jax
import jax.numpy as jnp
from jax import lax
from jax.experimental import pallas as pl
from jax.experimental.pallas import tpu as pltpu
from jax.experimental.pallas import tpu_sc as plsc

N_ENT = 100000
N_REL = 16
BATCH = 1024
K = 15
CH = 128
NPAD = 102400
C = NPAD // CH
NB = 2048
GRID = NPAD // NB
GPB = NB // CH
BIG = 1 << 30


def _sim_body(q_ref, e_ref, sim_ref, gm_ref):
    q = q_ref[...]
    e = e_ref[...]
    s = lax.dot_general(q, e, (((1,), (1,)), ((), ())),
                        preferred_element_type=jnp.float32)
    sim_ref[...] = s
    gm_ref[0] = s.reshape(BATCH, GPB, CH).max(axis=2)


def _chunksel_body(gm_ref, gidx_ref):
    g = gm_ref[...]
    iota_c = lax.broadcasted_iota(jnp.int32, (BATCH, C), 1)
    chs = []
    for _ in range(K):
        m = jnp.max(g, axis=1, keepdims=True)
        c = jnp.min(jnp.where(g == m, iota_c, BIG), axis=1, keepdims=True)
        chs.append(c)
        g = jnp.where(iota_c == c, jnp.float32(-1.0), g)
    lanes = lax.broadcasted_iota(jnp.int32, (BATCH, K * CH), 1)
    slot = lanes // CH
    within = lanes - slot * CH
    base = jnp.zeros((BATCH, K * CH), jnp.int32)
    for j in range(K):
        base = jnp.where(slot == j, chs[j], base)
    gidx_ref[...] = base * CH + within


def _final_body(cand_ref, gidx_ref, tv_ref, ti_ref):
    v = cand_ref[...]
    gidx = gidx_ref[...]
    lanes = lax.broadcasted_iota(jnp.int32, (BATCH, CH), 1)
    tv = jnp.zeros((BATCH, CH), jnp.float32)
    ti = jnp.zeros((BATCH, CH), jnp.int32)
    for j in range(K):
        m = jnp.max(v, axis=1, keepdims=True)
        gi = jnp.min(jnp.where(v == m, gidx, BIG), axis=1, keepdims=True)
        tv = jnp.where(lanes == j, m, tv)
        ti = jnp.where(lanes == j, gi, ti)
        v = jnp.where(gidx == gi, jnp.float32(-1.0), v)
    tv_ref[...] = tv
    ti_ref[...] = ti


def _sc_gather(table, idx):
    info = plsc.get_sparse_core_info()
    nw = info.num_cores * info.num_subcores
    b = idx.shape[0]
    bpw = b // nw
    mesh = plsc.VectorSubcoreMesh(core_axis_name="c", subcore_axis_name="s")

    @functools.partial(
        pl.kernel, mesh=mesh,
        out_type=jax.ShapeDtypeStruct((b, CH), jnp.float32),
        scratch_types=[
            pltpu.VMEM((bpw,), jnp.int32),
            pltpu.VMEM((bpw, CH), jnp.float32),
            pltpu.SemaphoreType.DMA,
        ],
    )
    def k(table_hbm, idx_hbm, out_hbm, idx_v, rows_v, sem):
        wid = lax.axis_index("s") * info.num_cores + lax.axis_index("c")
        base = wid * bpw
        pltpu.sync_copy(idx_hbm.at[pl.ds(base, bpw)], idx_v)
        pltpu.async_copy(table_hbm.at[idx_v], rows_v, sem).wait()
        pltpu.sync_copy(rows_v, out_hbm.at[pl.ds(base, bpw)])

    return k(table, idx)


def kernel(entity_vectors, query_entities, k):
    evp = jnp.pad(entity_vectors, ((0, NPAD - N_ENT), (0, 0)))
    qv = jnp.take(evp, query_entities, axis=0)

    sim, gm = pl.pallas_call(
        _sim_body,
        grid=(GRID,),
        in_specs=[
            pl.BlockSpec((BATCH, N_REL), lambda i: (0, 0)),
            pl.BlockSpec((NB, N_REL), lambda i: (i, 0)),
        ],
        out_specs=[
            pl.BlockSpec((BATCH, NB), lambda i: (0, i)),
            pl.BlockSpec((1, BATCH, GPB), lambda i: (i, 0, 0)),
        ],
        out_shape=[
            jax.ShapeDtypeStruct((BATCH, NPAD), jnp.float32),
            jax.ShapeDtypeStruct((GRID, BATCH, GPB), jnp.float32),
        ],
    )(qv, evp)

    gm2 = gm.transpose(1, 0, 2).reshape(BATCH, C)

    gidx = pl.pallas_call(
        _chunksel_body,
        out_shape=jax.ShapeDtypeStruct((BATCH, K * CH), jnp.int32),
    )(gm2)

    rowids = (jnp.arange(BATCH, dtype=jnp.int32)[:, None] * C
              + gidx[:, ::CH] // CH).reshape(-1)
    cand = _sc_gather(sim.reshape(BATCH * C, CH), rowids)

    tv, ti = pl.pallas_call(
        _final_body,
        out_shape=[
            jax.ShapeDtypeStruct((BATCH, CH), jnp.float32),
            jax.ShapeDtypeStruct((BATCH, CH), jnp.int32),
        ],
    )(cand.reshape(BATCH, K * CH), gidx)

    return tv[:, :K], ti[:, :K]


# speedup vs baseline: 5.8192x; 1.1073x over previous
import functools

import jax
import jax.numpy as jnp
from jax import lax
from jax.experimental import pallas as pl
from jax.experimental.pallas import tpu as pltpu
from jax.experimental.pallas import tpu_sc as plsc

N_ENT = 100000
N_REL = 16
BATCH = 1024
K = 15
CH = 128
NPAD = 102400
C = NPAD // CH
NB = 2048
GRID = NPAD // NB
GPB = NB // CH
BIG = 1 << 30


def _sim_body(q_ref, e_ref, sim_ref, gm_ref):
    q = q_ref[...]
    e = e_ref[...]
    s = lax.dot_general(q, e, (((1,), (1,)), ((), ())),
                        preferred_element_type=jnp.float32)
    sim_ref[...] = s
    st = lax.dot_general(e, q, (((1,), (1,)), ((), ())),
                         preferred_element_type=jnp.float32)
    gm_ref[0] = st.reshape(GPB, CH, BATCH).max(axis=1)


def _chunksel_body(gm_ref, ch_ref):
    g = gm_ref[...]
    iota_c = lax.broadcasted_iota(jnp.int32, (C, BATCH), 0)
    for j in range(16):
        m = jnp.max(g, axis=0, keepdims=True)
        c = jnp.min(jnp.where(g == m, iota_c, BIG), axis=0, keepdims=True)
        ch_ref[j, :] = c[0]
        g = jnp.where(iota_c == c, jnp.float32(-1.0), g)


def _final_body(cand_ref, ch_ref, tv_ref, ti_ref):
    v = cand_ref[...]
    ch = ch_ref[...]
    lanes = lax.broadcasted_iota(jnp.int32, (BATCH, K * CH), 1)
    slot = lanes // CH
    within = lanes - slot * CH
    base = jnp.zeros((BATCH, K * CH), jnp.int32)
    for j in range(K):
        base = jnp.where(slot == j, ch[:, j:j + 1], base)
    gidx = base * CH + within
    out_lanes = lax.broadcasted_iota(jnp.int32, (BATCH, CH), 1)
    tv = jnp.zeros((BATCH, CH), jnp.float32)
    ti = jnp.zeros((BATCH, CH), jnp.int32)
    for j in range(K):
        m = jnp.max(v, axis=1, keepdims=True)
        gi = jnp.min(jnp.where(v == m, gidx, BIG), axis=1, keepdims=True)
        tv = jnp.where(out_lanes == j, m, tv)
        ti = jnp.where(out_lanes == j, gi, ti)
        v = jnp.where(gidx == gi, jnp.float32(-1.0), v)
    tv_ref[...] = tv
    ti_ref[...] = ti


def _sc_gather(table, idx):
    info = plsc.get_sparse_core_info()
    nw = info.num_cores * info.num_subcores
    b = idx.shape[0]
    bpw = b // nw
    mesh = plsc.VectorSubcoreMesh(core_axis_name="c", subcore_axis_name="s")

    @functools.partial(
        pl.kernel, mesh=mesh,
        out_type=jax.ShapeDtypeStruct((b, CH), jnp.float32),
        scratch_types=[
            pltpu.VMEM((bpw,), jnp.int32),
            pltpu.VMEM((bpw, CH), jnp.float32),
            pltpu.SemaphoreType.DMA,
        ],
    )
    def k(table_hbm, idx_hbm, out_hbm, idx_v, rows_v, sem):
        wid = lax.axis_index("s") * info.num_cores + lax.axis_index("c")
        base = wid * bpw
        pltpu.sync_copy(idx_hbm.at[pl.ds(base, bpw)], idx_v)
        pltpu.async_copy(table_hbm.at[idx_v], rows_v, sem).wait()
        pltpu.sync_copy(rows_v, out_hbm.at[pl.ds(base, bpw)])

    return k(table, idx)


def kernel(entity_vectors, query_entities, k):
    evp = jnp.pad(entity_vectors, ((0, NPAD - N_ENT), (0, 0)))
    qv = jnp.take(evp, query_entities, axis=0)

    sim, gm = pl.pallas_call(
        _sim_body,
        grid=(GRID,),
        in_specs=[
            pl.BlockSpec((BATCH, N_REL), lambda i: (0, 0)),
            pl.BlockSpec((NB, N_REL), lambda i: (i, 0)),
        ],
        out_specs=[
            pl.BlockSpec((BATCH, NB), lambda i: (0, i)),
            pl.BlockSpec((1, GPB, BATCH), lambda i: (i, 0, 0)),
        ],
        out_shape=[
            jax.ShapeDtypeStruct((BATCH, NPAD), jnp.float32),
            jax.ShapeDtypeStruct((GRID, GPB, BATCH), jnp.float32),
        ],
    )(qv, evp)

    cht = pl.pallas_call(
        _chunksel_body,
        out_shape=jax.ShapeDtypeStruct((GPB, BATCH), jnp.int32),
    )(gm.reshape(C, BATCH))

    ch = cht.T
    rowids = (jnp.arange(BATCH, dtype=jnp.int32)[:, None] * C
              + ch[:, :K]).reshape(-1)
    cand = _sc_gather(sim.reshape(BATCH * C, CH), rowids)

    tv, ti = pl.pallas_call(
        _final_body,
        out_shape=[
            jax.ShapeDtypeStruct((BATCH, CH), jnp.float32),
            jax.ShapeDtypeStruct((BATCH, CH), jnp.int32),
        ],
    )(cand.reshape(BATCH, K * CH), ch)

    return tv[:, :K], ti[:, :K]


# speedup vs baseline: 11.8915x; 2.0435x over previous
import functools

import jax
import jax.numpy as jnp
from jax import lax
from jax.experimental import pallas as pl
from jax.experimental.pallas import tpu as pltpu
from jax.experimental.pallas import tpu_sc as plsc

N_ENT = 100000
N_REL = 16
BATCH = 1024
K = 15
CH = 128
NPAD = 102400
C = NPAD // CH
NB = 2048
GRID = NPAD // NB
GPB = NB // CH
BIG = 1 << 30


def _sim_body(q_ref, e_ref, sim_ref, gm_ref):
    q = q_ref[...]
    e = e_ref[...]
    s = lax.dot_general(q, e, (((1,), (1,)), ((), ())),
                        preferred_element_type=jnp.float32)
    for c in range(GPB):
        sim_ref[c] = s[:, c * CH:(c + 1) * CH]
    st = lax.dot_general(e, q, (((1,), (1,)), ((), ())),
                         preferred_element_type=jnp.float32)
    gm_ref[0] = st.reshape(GPB, CH, BATCH).max(axis=1)


def _chunksel_body(gm_ref, ch_ref):
    g = gm_ref[...]
    iota_c = lax.broadcasted_iota(jnp.int32, (C, BATCH), 0)
    for j in range(16):
        m = jnp.max(g, axis=0, keepdims=True)
        c = jnp.min(jnp.where(g == m, iota_c, BIG), axis=0, keepdims=True)
        ch_ref[j, :] = c[0]
        g = jnp.where(iota_c == c, jnp.float32(-1.0), g)


def _final_body(cand_ref, ch_ref, tv_ref, ti_ref):
    v = cand_ref[...]
    ch = ch_ref[...]
    lanes = lax.broadcasted_iota(jnp.int32, (BATCH, K * CH), 1)
    slot = lanes // CH
    within = lanes - slot * CH
    base = jnp.zeros((BATCH, K * CH), jnp.int32)
    for j in range(K):
        base = jnp.where(slot == j, ch[:, j:j + 1], base)
    gidx = base * CH + within
    out_lanes = lax.broadcasted_iota(jnp.int32, (BATCH, CH), 1)
    tv = jnp.zeros((BATCH, CH), jnp.float32)
    ti = jnp.zeros((BATCH, CH), jnp.int32)
    for j in range(K):
        m = jnp.max(v, axis=1, keepdims=True)
        gi = jnp.min(jnp.where(v == m, gidx, BIG), axis=1, keepdims=True)
        tv = jnp.where(out_lanes == j, m, tv)
        ti = jnp.where(out_lanes == j, gi, ti)
        v = jnp.where(gidx == gi, jnp.float32(-1.0), v)
    tv_ref[...] = tv
    ti_ref[...] = ti


def _sc_gather(table, idx):
    info = plsc.get_sparse_core_info()
    nw = info.num_cores * info.num_subcores
    b = idx.shape[0]
    bpw = b // nw
    mesh = plsc.VectorSubcoreMesh(core_axis_name="c", subcore_axis_name="s")

    @functools.partial(
        pl.kernel, mesh=mesh,
        out_type=jax.ShapeDtypeStruct((b, CH), jnp.float32),
        scratch_types=[
            pltpu.VMEM((bpw,), jnp.int32),
            pltpu.VMEM((bpw, CH), jnp.float32),
            pltpu.SemaphoreType.DMA,
        ],
    )
    def k(table_hbm, idx_hbm, out_hbm, idx_v, rows_v, sem):
        wid = lax.axis_index("s") * info.num_cores + lax.axis_index("c")
        base = wid * bpw
        pltpu.sync_copy(idx_hbm.at[pl.ds(base, bpw)], idx_v)
        pltpu.async_copy(table_hbm.at[idx_v], rows_v, sem).wait()
        pltpu.sync_copy(rows_v, out_hbm.at[pl.ds(base, bpw)])

    return k(table, idx)


def kernel(entity_vectors, query_entities, k):
    evp = jnp.pad(entity_vectors, ((0, NPAD - N_ENT), (0, 0)))
    qv = jnp.take(evp, query_entities, axis=0)

    sim, gm = pl.pallas_call(
        _sim_body,
        grid=(GRID,),
        in_specs=[
            pl.BlockSpec((BATCH, N_REL), lambda i: (0, 0)),
            pl.BlockSpec((NB, N_REL), lambda i: (i, 0)),
        ],
        out_specs=[
            pl.BlockSpec((GPB, BATCH, CH), lambda i: (i, 0, 0)),
            pl.BlockSpec((1, GPB, BATCH), lambda i: (i, 0, 0)),
        ],
        out_shape=[
            jax.ShapeDtypeStruct((C, BATCH, CH), jnp.float32),
            jax.ShapeDtypeStruct((GRID, GPB, BATCH), jnp.float32),
        ],
    )(qv, evp)

    cht = pl.pallas_call(
        _chunksel_body,
        out_shape=jax.ShapeDtypeStruct((GPB, BATCH), jnp.int32),
    )(gm.reshape(C, BATCH))

    ch = cht.T
    rowids = (ch[:, :K] * BATCH
              + jnp.arange(BATCH, dtype=jnp.int32)[:, None]).reshape(-1)
    cand = _sc_gather(sim.reshape(C * BATCH, CH), rowids)

    tv, ti = pl.pallas_call(
        _final_body,
        out_shape=[
            jax.ShapeDtypeStruct((BATCH, CH), jnp.float32),
            jax.ShapeDtypeStruct((BATCH, CH), jnp.int32),
        ],
    )(cand.reshape(BATCH, K * CH), ch)

    return tv[:, :K], ti[:, :K]


# speedup vs baseline: 11.9051x; 1.0011x over previous
import functools

import jax
import jax.numpy as jnp
from jax import lax
from jax.experimental import pallas as pl
from jax.experimental.pallas import tpu as pltpu
from jax.experimental.pallas import tpu_sc as plsc

N_ENT = 100000
N_REL = 16
BATCH = 1024
K = 15
CH = 128
NPAD = 102400
C = NPAD // CH
NB = 2048
GRID = NPAD // NB
GPB = NB // CH
BIG = 1 << 30


def _sim_body(q_ref, e_ref, sim_ref, gm_ref):
    q = q_ref[...]
    e = e_ref[...]
    s = lax.dot_general(q, e, (((1,), (1,)), ((), ())),
                        preferred_element_type=jnp.float32)
    for c in range(GPB):
        sim_ref[c] = s[:, c * CH:(c + 1) * CH]
    st = lax.dot_general(e, q, (((1,), (1,)), ((), ())),
                         preferred_element_type=jnp.float32)
    gm_ref[0] = st.reshape(GPB, CH, BATCH).max(axis=1)


def _chunksel_body(gm_ref, ch_ref):
    g = gm_ref[...]
    iota_c = lax.broadcasted_iota(jnp.int32, (C, BATCH), 0)
    for j in range(16):
        m = jnp.max(g, axis=0, keepdims=True)
        c = jnp.min(jnp.where(g == m, iota_c, BIG), axis=0, keepdims=True)
        ch_ref[j, :] = c[0]
        g = jnp.where(iota_c == c, jnp.float32(-1.0), g)


def _final_body(cand_ref, ch_ref, tv_ref, ti_ref):
    v = cand_ref[...]
    ch = ch_ref[...]
    lanes = lax.broadcasted_iota(jnp.int32, (BATCH, K * CH), 1)
    slot = lanes // CH
    within = lanes - slot * CH
    base = jnp.zeros((BATCH, K * CH), jnp.int32)
    for j in range(K):
        base = jnp.where(slot == j, ch[:, j:j + 1], base)
    gidx = base * CH + within
    out_lanes = lax.broadcasted_iota(jnp.int32, (BATCH, K), 1)
    tv = jnp.zeros((BATCH, K), jnp.float32)
    ti = jnp.zeros((BATCH, K), jnp.int32)
    for j in range(K):
        m = jnp.max(v, axis=1, keepdims=True)
        gi = jnp.min(jnp.where(v == m, gidx, BIG), axis=1, keepdims=True)
        tv = jnp.where(out_lanes == j, m, tv)
        ti = jnp.where(out_lanes == j, gi, ti)
        v = jnp.where(gidx == gi, jnp.float32(-1.0), v)
    tv_ref[...] = tv
    ti_ref[...] = ti


def _sc_gather(table, idx):
    info = plsc.get_sparse_core_info()
    nw = info.num_cores * info.num_subcores
    b = idx.shape[0]
    bpw = b // nw
    mesh = plsc.VectorSubcoreMesh(core_axis_name="c", subcore_axis_name="s")

    @functools.partial(
        pl.kernel, mesh=mesh,
        out_type=jax.ShapeDtypeStruct((b, CH), jnp.float32),
        scratch_types=[
            pltpu.VMEM((bpw,), jnp.int32),
            pltpu.VMEM((bpw, CH), jnp.float32),
            pltpu.SemaphoreType.DMA,
        ],
    )
    def k(table_hbm, idx_hbm, out_hbm, idx_v, rows_v, sem):
        wid = lax.axis_index("s") * info.num_cores + lax.axis_index("c")
        base = wid * bpw
        pltpu.sync_copy(idx_hbm.at[pl.ds(base, bpw)], idx_v)
        pltpu.async_copy(table_hbm.at[idx_v], rows_v, sem).wait()
        pltpu.sync_copy(rows_v, out_hbm.at[pl.ds(base, bpw)])

    return k(table, idx)


def kernel(entity_vectors, query_entities, k):
    evp = jnp.pad(entity_vectors, ((0, NPAD - N_ENT), (0, 0)))
    qv = jnp.take(evp, query_entities, axis=0)

    sim, gm = pl.pallas_call(
        _sim_body,
        grid=(GRID,),
        in_specs=[
            pl.BlockSpec((BATCH, N_REL), lambda i: (0, 0)),
            pl.BlockSpec((NB, N_REL), lambda i: (i, 0)),
        ],
        out_specs=[
            pl.BlockSpec((GPB, BATCH, CH), lambda i: (i, 0, 0)),
            pl.BlockSpec((1, GPB, BATCH), lambda i: (i, 0, 0)),
        ],
        out_shape=[
            jax.ShapeDtypeStruct((C, BATCH, CH), jnp.float32),
            jax.ShapeDtypeStruct((GRID, GPB, BATCH), jnp.float32),
        ],
    )(qv, evp)

    cht = pl.pallas_call(
        _chunksel_body,
        out_shape=jax.ShapeDtypeStruct((GPB, BATCH), jnp.int32),
    )(gm.reshape(C, BATCH))

    ch = cht.T
    rowids = (ch[:, :K] * BATCH
              + jnp.arange(BATCH, dtype=jnp.int32)[:, None]).reshape(-1)
    cand = _sc_gather(sim.reshape(C * BATCH, CH), rowids)

    tv, ti = pl.pallas_call(
        _final_body,
        out_shape=[
            jax.ShapeDtypeStruct((BATCH, K), jnp.float32),
            jax.ShapeDtypeStruct((BATCH, K), jnp.int32),
        ],
    )(cand.reshape(BATCH, K * CH), ch)

    return tv, ti
